# baseline (speedup 1.0000x reference)
import functools

import jax
import jax.numpy as jnp
from jax import lax
from jax.experimental import layout as jex_layout
from jax.experimental import pallas as pl
from jax.experimental.pallas import tpu as pltpu
from jax.experimental.pallas import tpu_sc as plsc

_V = 10000
_E = 40000
_D = 64

_NC = 2
_NS = 16
_L = 16

_OWN = _E // _NS
_WIN = 2512
_CHUNK = 128
_NFULL = 19
_TAIL_OFF = _WIN - _CHUNK
_VHALF = _V // _NC
_DUMMY = _VHALF
_ACC_ROWS = 5200
_ZROWS = _ACC_ROWS // _NS
_OUT_ROWS = 312
_HBUF = 384

_CHUNK_OFF = [k * _CHUNK for k in range(_NFULL)] + [_TAIL_OFF]
_PIECES = [
    (0, _HBUF, range(0, 3)),
    (_HBUF, _HBUF, range(3, 6)),
    (2 * _HBUF, _HBUF, range(6, 9)),
    (3 * _HBUF, _HBUF, range(9, 12)),
    (4 * _HBUF, _HBUF, range(12, 15)),
    (5 * _HBUF, _HBUF, range(15, 18)),
    (6 * _HBUF, _WIN - 6 * _HBUF, range(18, _NFULL + 1)),
]

_mesh = plsc.VectorSubcoreMesh(
    core_axis_name="c", subcore_axis_name="s",
    num_cores=_NC, num_subcores=_NS)


@functools.partial(
    pl.kernel,
    out_type=jax.ShapeDtypeStruct((_V, 2 * _D), jnp.float32),
    mesh=_mesh,
    scratch_types=[
        pltpu.VMEM((_WIN,), jnp.int32),
        pltpu.VMEM((_NFULL + 1, _CHUNK), jnp.int32),
        pltpu.VMEM((_HBUF, _D), jnp.float32),
        pltpu.VMEM((_HBUF, _D), jnp.float32),
        pltpu.VMEM((_HBUF, _D), jnp.float32),
        pltpu.VMEM((_HBUF, _D), jnp.float32),
        pltpu.VMEM_SHARED((_ACC_ROWS, _D), jnp.float32),
        pltpu.SemaphoreType.DMA,
        pltpu.SemaphoreType.DMA,
        pltpu.SemaphoreType.DMA,
        pltpu.SemaphoreType.DMA,
    ],
    compiler_params=pltpu.CompilerParams(use_tc_tiling_on_sc=False),
)
def _aggr_sum(h_hbm, idx_hbm, out_hbm,
              ibuf1d, ibuf, hbuf0, hbuf1, hbuf2, hbuf3, acc,
              sem_i, sem_h, sem_z, sem_s):
    c = lax.axis_index("c")
    s = lax.axis_index("s")
    lo_e = s * _OWN
    w = (lo_e // _L) * _L
    c_lo = c * _VHALF

    hbufs = [hbuf0, hbuf1, hbuf2, hbuf3]
    nbuf = len(hbufs)

    def _load(p):
        off, rows, _ = _PIECES[p]
        return pltpu.async_copy(
            h_hbm.at[pl.ds(w + off, rows), pl.ds(0, _D)],
            hbufs[p % nbuf].at[pl.ds(0, rows)], sem_h)

    d_idx = pltpu.async_copy(idx_hbm.at[pl.ds(w, _WIN)], ibuf1d, sem_i)
    d_h = [None] * len(_PIECES)
    for p in range(nbuf):
        d_h[p] = _load(p)
    d_z = pltpu.async_copy(
        h_hbm.at[pl.ds(s * _ZROWS, _ZROWS), pl.ds(_D, _D)],
        acc.at[pl.ds(s * _ZROWS, _ZROWS)], sem_z)

    d_idx.wait()
    iota = lax.broadcasted_iota(jnp.int32, (_L,), 0)

    def _remap(i, _):
        k = i // (_CHUNK // _L)
        j = i - k * (_CHUNK // _L)
        off = jnp.where(k < _NFULL, k * _CHUNK, _TAIL_OFF)
        g_lo = jnp.where(k < _NFULL, lo_e, w + _NFULL * _CHUNK)
        g = w + off + j * _L + iota
        v = ibuf1d[pl.ds(off + j * _L, _L)]
        own = (g >= g_lo) & (g < lo_e + _OWN)
        keep = own & (v >= c_lo) & (v < c_lo + _VHALF)
        dummy = _DUMMY + (g & (_L * 8 - 1)) + ((g >> 7) & 1) * (_L * 4)
        ibuf[k, pl.ds(j * _L, _L)] = jnp.where(keep, v - c_lo, dummy)
        return 0

    lax.fori_loop(0, (_NFULL + 1) * (_CHUNK // _L), _remap, 0)

    d_z.wait()
    plsc.subcore_barrier()

    d_sc = [None] * len(_PIECES)
    for p, (off, rows, chunks) in enumerate(_PIECES):
        d_h[p].wait()
        d_sc[p] = [
            pltpu.async_copy(
                hbufs[p % nbuf].at[pl.ds(_CHUNK_OFF[k] - off, _CHUNK)],
                acc.at[ibuf.at[k]], sem_s, add=True)
            for k in chunks
        ]
        if p + nbuf < len(_PIECES):
            for d in d_sc[p]:
                d.wait()
            d_sc[p] = []
            d_h[p + nbuf] = _load(p + nbuf)
    for descs in d_sc:
        for d in descs:
            d.wait()

    plsc.subcore_barrier()

    r = s * _OUT_ROWS
    pltpu.sync_copy(acc.at[pl.ds(r, _OUT_ROWS)],
                    out_hbm.at[pl.ds(c_lo + r, _OUT_ROWS), pl.ds(0, _D)])

    @pl.when(s == 0)
    def _():
        tail = _NS * _OUT_ROWS
        pltpu.sync_copy(
            acc.at[pl.ds(tail, _VHALF - tail)],
            out_hbm.at[pl.ds(c_lo + tail, _VHALF - tail), pl.ds(0, _D)])


def kernel(H, X_node):
    def _linear(x):
        lay = jex_layout.Layout(tuple(range(x.ndim)), tiling=())
        return jex_layout.with_layout_constraint(x, lay)

    h128 = _linear(jnp.pad(H, ((0, 0), (0, _D))))
    out128 = _linear(_aggr_sum(h128, _linear(X_node.astype(jnp.int32))))
    return out128[:, :_D]


# speedup vs baseline: 1.0031x; 1.0031x over previous
import functools

import jax
import jax.numpy as jnp
from jax import lax
from jax.experimental import layout as jex_layout
from jax.experimental import pallas as pl
from jax.experimental.pallas import tpu as pltpu
from jax.experimental.pallas import tpu_sc as plsc

_V = 10000
_E = 40000
_D = 64

_NC = 2
_NS = 16
_L = 16

_OWN = _E // _NS
_WIN = 2512
_CHUNK = 128
_NFULL = 19
_TAIL_OFF = _WIN - _CHUNK
_VHALF = _V // _NC
_DUMMY = _VHALF
_ACC_ROWS = 5184
_ZROWS = _ACC_ROWS // _NS
_OUT_ROWS = 312
_HBUF = 384

_CHUNK_OFF = [k * _CHUNK for k in range(_NFULL)] + [_TAIL_OFF]
_PIECES = [
    (0, _HBUF, range(0, 3)),
    (_HBUF, _HBUF, range(3, 6)),
    (2 * _HBUF, _HBUF, range(6, 9)),
    (3 * _HBUF, _HBUF, range(9, 12)),
    (4 * _HBUF, _HBUF, range(12, 15)),
    (5 * _HBUF, _HBUF, range(15, 18)),
    (6 * _HBUF, _WIN - 6 * _HBUF, range(18, _NFULL + 1)),
]

_mesh = plsc.VectorSubcoreMesh(
    core_axis_name="c", subcore_axis_name="s",
    num_cores=_NC, num_subcores=_NS)


@functools.partial(
    pl.kernel,
    out_type=jax.ShapeDtypeStruct((_V, 2 * _D), jnp.float32),
    mesh=_mesh,
    scratch_types=[
        pltpu.VMEM((_WIN,), jnp.int32),
        pltpu.VMEM((_NFULL + 1, _CHUNK), jnp.int32),
        pltpu.VMEM((_HBUF, _D), jnp.float32),
        pltpu.VMEM((_HBUF, _D), jnp.float32),
        pltpu.VMEM((_HBUF, _D), jnp.float32),
        pltpu.VMEM((_HBUF, _D), jnp.float32),
        pltpu.VMEM_SHARED((_ACC_ROWS, _D), jnp.float32),
        pltpu.SemaphoreType.DMA,
        pltpu.SemaphoreType.DMA,
        pltpu.SemaphoreType.DMA,
        pltpu.SemaphoreType.DMA,
    ],
    compiler_params=pltpu.CompilerParams(use_tc_tiling_on_sc=False),
)
def _aggr_sum(h_hbm, idx_hbm, out_hbm,
              ibuf1d, ibuf, hbuf0, hbuf1, hbuf2, hbuf3, acc,
              sem_i, sem_h, sem_z, sem_s):
    c = lax.axis_index("c")
    s = lax.axis_index("s")
    lo_e = s * _OWN
    w = (lo_e // _L) * _L
    c_lo = c * _VHALF

    hbufs = [hbuf0, hbuf1, hbuf2, hbuf3]
    nbuf = len(hbufs)

    def _load(p):
        off, rows, _ = _PIECES[p]
        return pltpu.async_copy(
            h_hbm.at[pl.ds(w + off, rows), pl.ds(0, _D)],
            hbufs[p % nbuf].at[pl.ds(0, rows)], sem_h)

    d_idx = pltpu.async_copy(idx_hbm.at[pl.ds(w, _WIN)], ibuf1d, sem_i)
    d_h = [None] * len(_PIECES)
    for p in range(nbuf):
        d_h[p] = _load(p)
    d_z = pltpu.async_copy(
        h_hbm.at[pl.ds(s * _ZROWS, _ZROWS), pl.ds(_D, _D)],
        acc.at[pl.ds(s * _ZROWS, _ZROWS)], sem_z)

    d_idx.wait()
    iota = lax.broadcasted_iota(jnp.int32, (_L,), 0)

    def _remap(i, _):
        k = i // (_CHUNK // _L)
        j = i - k * (_CHUNK // _L)
        off = jnp.where(k < _NFULL, k * _CHUNK, _TAIL_OFF)
        g_lo = jnp.where(k < _NFULL, lo_e, w + _NFULL * _CHUNK)
        g = w + off + j * _L + iota
        v = ibuf1d[pl.ds(off + j * _L, _L)]
        own = (g >= g_lo) & (g < lo_e + _OWN)
        keep = own & (v >= c_lo) & (v < c_lo + _VHALF)
        dummy = _DUMMY + (g & (_L * 8 - 1))
        ibuf[k, pl.ds(j * _L, _L)] = jnp.where(keep, v - c_lo, dummy)
        return 0

    lax.fori_loop(0, (_NFULL + 1) * (_CHUNK // _L), _remap, 0)

    d_z.wait()
    plsc.subcore_barrier()

    d_sc = [None] * len(_PIECES)
    for p, (off, rows, chunks) in enumerate(_PIECES):
        d_h[p].wait()
        d_sc[p] = [
            pltpu.async_copy(
                hbufs[p % nbuf].at[pl.ds(_CHUNK_OFF[k] - off, _CHUNK)],
                acc.at[ibuf.at[k]], sem_s, add=True)
            for k in chunks
        ]
        if p + nbuf < len(_PIECES):
            for d in d_sc[p]:
                d.wait()
            d_sc[p] = []
            d_h[p + nbuf] = _load(p + nbuf)
    for descs in d_sc:
        for d in descs:
            d.wait()

    plsc.subcore_barrier()

    r = s * _OUT_ROWS
    pltpu.sync_copy(acc.at[pl.ds(r, _OUT_ROWS)],
                    out_hbm.at[pl.ds(c_lo + r, _OUT_ROWS), pl.ds(0, _D)])

    @pl.when(s == 0)
    def _():
        tail = _NS * _OUT_ROWS
        pltpu.sync_copy(
            acc.at[pl.ds(tail, _VHALF - tail)],
            out_hbm.at[pl.ds(c_lo + tail, _VHALF - tail), pl.ds(0, _D)])


def kernel(H, X_node):
    def _linear(x):
        lay = jex_layout.Layout(tuple(range(x.ndim)), tiling=())
        return jex_layout.with_layout_constraint(x, lay)

    h128 = _linear(jnp.pad(H, ((0, 0), (0, _D))))
    out128 = _linear(_aggr_sum(h128, _linear(X_node.astype(jnp.int32))))
    return out128[:, :_D]
